# Initial kernel scaffold; baseline (speedup 1.0000x reference)
import jax
import jax.numpy as jnp
from jax.experimental import pallas as pl


def kernel(batch, edge_index_c2, edge_index_c3, edge_index_c4, fc_b, fc_w, l0_c2_b_in, l0_c2_b_out, l0_c2_w_in, l0_c2_w_out, l0_c3_b_in, l0_c3_b_out, l0_c3_w_in, l0_c3_w_out, l0_c4_b_in, l0_c4_b_out, l0_c4_w_in, l0_c4_w_out, l1_c2_b_in, l1_c2_b_out, l1_c2_w_in, l1_c2_w_out, l1_c3_b_in, l1_c3_b_out, l1_c3_w_in, l1_c3_w_out, l1_c4_b_in, l1_c4_b_out, l1_c4_w_in, l1_c4_w_out, x):
    raise NotImplementedError("write your pallas kernel here")


# speedup vs baseline: 11.9330x; 11.9330x over previous
import functools

import jax
import jax.numpy as jnp
from jax import lax
from jax.experimental import pallas as pl
from jax.experimental.pallas import tpu as pltpu
from jax.experimental.pallas import tpu_sc as plsc

N = 10000
E = 320000
NSUB = 16
LANES = 128
ROWS_PER_SUB = 160
ROWS_PAD = ROWS_PER_SUB * NSUB
E_PAD = ROWS_PAD * LANES
N_PAD = 10112
SLICE = N_PAD // NSUB
W = 8

_f32 = jnp.float32


def _tc1_body(x_ref, w_ref, b_ref, o2_ref, o3_ref, o4_ref):
    h = jnp.dot(x_ref[...], w_ref[...], preferred_element_type=_f32)
    h = jnp.maximum(h + b_ref[...], 0.0)
    o2_ref[...] = h[:, 0:8]
    o3_ref[...] = h[:, 8:16]
    o4_ref[...] = h[:, 16:24]


def _tc2_body(h2_ref, h3_ref, h4_ref, wo_ref, bo_ref, wi_ref, bi_ref,
              o2_ref, o3_ref, o4_ref):
    o2 = jnp.maximum(jnp.dot(h2_ref[...], wo_ref[0], preferred_element_type=_f32) + bo_ref[0:1, :], 0.0)
    o3 = jnp.maximum(jnp.dot(h3_ref[...], wo_ref[1], preferred_element_type=_f32) + bo_ref[1:2, :], 0.0)
    o4 = jnp.maximum(jnp.dot(h4_ref[...], wo_ref[2], preferred_element_type=_f32) + bo_ref[2:3, :], 0.0)
    feat = jnp.concatenate([o2, o3, o4], axis=1)
    h = jnp.dot(feat, wi_ref[...], preferred_element_type=_f32)
    h = jnp.maximum(h + bi_ref[...], 0.0)
    o2_ref[...] = h[:, 0:8]
    o3_ref[...] = h[:, 8:16]
    o4_ref[...] = h[:, 16:24]


def _tc3_body(h2_ref, h3_ref, h4_ref, wo_ref, bo_ref, fcw_ref, fcb_ref, out_ref):
    o2 = jnp.maximum(jnp.dot(h2_ref[...], wo_ref[0], preferred_element_type=_f32) + bo_ref[0:1, :], 0.0)
    o3 = jnp.maximum(jnp.dot(h3_ref[...], wo_ref[1], preferred_element_type=_f32) + bo_ref[1:2, :], 0.0)
    o4 = jnp.maximum(jnp.dot(h4_ref[...], wo_ref[2], preferred_element_type=_f32) + bo_ref[2:3, :], 0.0)
    feat = jnp.concatenate([o2, o3, o4], axis=1)
    out_ref[...] = jnp.dot(feat, fcw_ref[...], preferred_element_type=_f32) + fcb_ref[...]


def _tc_call(body, n_out, *args):
    outs = [jax.ShapeDtypeStruct((N_PAD, W), _f32)] * n_out
    return pl.pallas_call(body, out_shape=outs)(*args)


_MESH = plsc.VectorSubcoreMesh(core_axis_name="c", subcore_axis_name="s")


@functools.partial(
    pl.kernel,
    out_type=(jax.ShapeDtypeStruct((N_PAD, W), _f32),) * 3,
    mesh=_MESH,
    scratch_types=[
        pltpu.VMEM_SHARED((N_PAD, W), _f32),
        pltpu.VMEM((ROWS_PER_SUB, LANES), jnp.int32),
        pltpu.VMEM((ROWS_PER_SUB, LANES), jnp.int32),
        pltpu.VMEM((LANES, W), _f32),
        pltpu.VMEM((LANES, W), _f32),
        pltpu.SemaphoreType.DMA,
    ],
    compiler_params=pltpu.CompilerParams(use_tc_tiling_on_sc=False),
)
def _mp_kernel(h2_hbm, h3_hbm, h4_hbm,
               s2_hbm, d2_hbm, s3_hbm, d3_hbm, s4_hbm, d4_hbm,
               zeros_hbm,
               out2_hbm, out3_hbm, out4_hbm,
               acc, src_v, dst_v, rows_a, rows_b, sem):
    cid = lax.axis_index("c")
    sid = lax.axis_index("s")
    row_lo = sid * ROWS_PER_SUB
    tab_lo = sid * SLICE

    def my_tab(ref):
        return ref.at[pl.ds(tab_lo, SLICE)]

    pltpu.sync_copy(my_tab(zeros_hbm), my_tab(acc))
    plsc.subcore_barrier()

    def wait_gather(buf):
        pltpu.make_async_copy(zeros_hbm.at[pl.ds(0, LANES)], buf, sem).wait()

    def run_pattern(h0_hbm, src_hbm, dst_hbm, out_hbm, length):
        pltpu.sync_copy(src_hbm.at[pl.ds(row_lo, ROWS_PER_SUB)], src_v)
        pltpu.sync_copy(dst_hbm.at[pl.ds(row_lo, ROWS_PER_SUB)], dst_v)
        for r in range(length):
            cur = h0_hbm if r == 0 else out_hbm
            pltpu.async_copy(cur.at[src_v.at[0]], rows_a, sem)

            def pair_body(k, _):
                j = 2 * k
                pltpu.async_copy(cur.at[src_v.at[j + 1]], rows_b, sem)
                wait_gather(rows_a)
                pltpu.sync_copy(rows_a, acc.at[dst_v.at[j]], add=True)
                pltpu.async_copy(cur.at[src_v.at[j + 2]], rows_a, sem)
                wait_gather(rows_b)
                pltpu.sync_copy(rows_b, acc.at[dst_v.at[j + 1]], add=True)
                return 0

            lax.fori_loop(0, ROWS_PER_SUB // 2 - 1, pair_body, 0)
            pltpu.async_copy(cur.at[src_v.at[ROWS_PER_SUB - 1]], rows_b, sem)
            wait_gather(rows_a)
            pltpu.sync_copy(rows_a, acc.at[dst_v.at[ROWS_PER_SUB - 2]], add=True)
            wait_gather(rows_b)
            pltpu.sync_copy(rows_b, acc.at[dst_v.at[ROWS_PER_SUB - 1]], add=True)
            plsc.subcore_barrier()
            pltpu.sync_copy(my_tab(acc), my_tab(out_hbm))
            pltpu.sync_copy(my_tab(zeros_hbm), my_tab(acc))
            plsc.subcore_barrier()

    @pl.when(cid == 0)
    def _():
        run_pattern(h4_hbm, s4_hbm, d4_hbm, out4_hbm, 4)
        plsc.subcore_barrier()
        plsc.subcore_barrier()

    @pl.when(cid == 1)
    def _():
        run_pattern(h2_hbm, s2_hbm, d2_hbm, out2_hbm, 2)
        run_pattern(h3_hbm, s3_hbm, d3_hbm, out3_hbm, 3)


def _prep_edges(edge_index):
    src = jnp.full((E_PAD,), N, jnp.int32).at[:E].set(edge_index[0])
    dst = jnp.full((E_PAD,), N, jnp.int32).at[:E].set(edge_index[1])
    return src.reshape(ROWS_PAD, LANES), dst.reshape(ROWS_PAD, LANES)


def _pad_cols(w, total):
    return jnp.pad(w, ((0, 0), (0, total - w.shape[1])))


def _pad_rows(w, total):
    return jnp.pad(w, ((0, total - w.shape[0]), (0, 0)))


def kernel(batch, edge_index_c2, edge_index_c3, edge_index_c4, fc_b, fc_w,
           l0_c2_b_in, l0_c2_b_out, l0_c2_w_in, l0_c2_w_out,
           l0_c3_b_in, l0_c3_b_out, l0_c3_w_in, l0_c3_w_out,
           l0_c4_b_in, l0_c4_b_out, l0_c4_w_in, l0_c4_w_out,
           l1_c2_b_in, l1_c2_b_out, l1_c2_w_in, l1_c2_w_out,
           l1_c3_b_in, l1_c3_b_out, l1_c3_w_in, l1_c3_w_out,
           l1_c4_b_in, l1_c4_b_out, l1_c4_w_in, l1_c4_w_out,
           x):
    del batch

    x_pad = jnp.pad(x, ((0, N_PAD - N), (0, 0)))
    zeros = jnp.zeros((N_PAD, W), _f32)

    s2, d2 = _prep_edges(edge_index_c2)
    s3, d3 = _prep_edges(edge_index_c3)
    s4, d4 = _prep_edges(edge_index_c4)

    w0 = jnp.concatenate([_pad_cols(l0_c2_w_in, W),
                          _pad_cols(l0_c3_w_in, W),
                          _pad_cols(l0_c4_w_in, W)], axis=1)
    b0 = jnp.concatenate([jnp.pad(l0_c2_b_in, (0, W - 5)),
                          jnp.pad(l0_c3_b_in, (0, W - 5)),
                          jnp.pad(l0_c4_b_in, (0, W - 5))]).reshape(1, 3 * W)
    h2, h3, h4 = _tc_call(_tc1_body, 3, x_pad, w0, b0)

    m2, m3, m4 = _mp_kernel(h2, h3, h4, s2, d2, s3, d3, s4, d4, zeros)

    wo0 = jnp.stack([_pad_rows(l0_c2_w_out, W),
                     _pad_rows(l0_c3_w_out, W),
                     _pad_rows(l0_c4_w_out, W)])
    bo0 = jnp.stack([l0_c2_b_out, l0_c3_b_out, l0_c4_b_out])
    wi1 = jnp.concatenate([_pad_cols(l1_c2_w_in, W),
                           _pad_cols(l1_c3_w_in, W),
                           _pad_cols(l1_c4_w_in, W)], axis=1)
    bi1 = jnp.concatenate([jnp.pad(l1_c2_b_in, (0, W - 5)),
                           jnp.pad(l1_c3_b_in, (0, W - 5)),
                           jnp.pad(l1_c4_b_in, (0, W - 5))]).reshape(1, 3 * W)
    g2, g3, g4 = pl.pallas_call(
        _tc2_body, out_shape=[jax.ShapeDtypeStruct((N_PAD, W), _f32)] * 3,
    )(m2, m3, m4, wo0, bo0, wi1, bi1)

    n2, n3, n4 = _mp_kernel(g2, g3, g4, s2, d2, s3, d3, s4, d4, zeros)

    wo1 = jnp.stack([_pad_rows(l1_c2_w_out, W),
                     _pad_rows(l1_c3_w_out, W),
                     _pad_rows(l1_c4_w_out, W)])
    bo1 = jnp.stack([l1_c2_b_out, l1_c3_b_out, l1_c4_b_out])
    out = pl.pallas_call(
        _tc3_body, out_shape=jax.ShapeDtypeStruct((N_PAD, 10), _f32),
    )(n2, n3, n4, wo1, bo1, fc_w, fc_b.reshape(1, 10))
    return out[:N]


# speedup vs baseline: 17.6342x; 1.4778x over previous
import functools

import jax
import jax.numpy as jnp
from jax import lax
from jax.experimental import pallas as pl
from jax.experimental.pallas import tpu as pltpu
from jax.experimental.pallas import tpu_sc as plsc

N = 10000
E = 320000
NSUB = 16
LANES = 128
ROWS_PER_SUB = 160
ROWS_PAD = ROWS_PER_SUB * NSUB
E_PAD = ROWS_PAD * LANES
N_PAD = 10112
SLICE = N_PAD // NSUB
W = 8
BLK = 8
NBLK = ROWS_PER_SUB // BLK

_f32 = jnp.float32


def _tc1_body(x_ref, w_ref, b_ref, o2_ref, o3_ref, o4_ref):
    h = jnp.dot(x_ref[...], w_ref[...], preferred_element_type=_f32)
    h = jnp.maximum(h + b_ref[...], 0.0)
    o2_ref[...] = h[:, 0:8]
    o3_ref[...] = h[:, 8:16]
    o4_ref[...] = h[:, 16:24]


def _tc2_body(h2_ref, h3_ref, h4_ref, wo_ref, bo_ref, wi_ref, bi_ref,
              o2_ref, o3_ref, o4_ref):
    o2 = jnp.maximum(jnp.dot(h2_ref[...], wo_ref[0], preferred_element_type=_f32) + bo_ref[0:1, :], 0.0)
    o3 = jnp.maximum(jnp.dot(h3_ref[...], wo_ref[1], preferred_element_type=_f32) + bo_ref[1:2, :], 0.0)
    o4 = jnp.maximum(jnp.dot(h4_ref[...], wo_ref[2], preferred_element_type=_f32) + bo_ref[2:3, :], 0.0)
    feat = jnp.concatenate([o2, o3, o4], axis=1)
    h = jnp.dot(feat, wi_ref[...], preferred_element_type=_f32)
    h = jnp.maximum(h + bi_ref[...], 0.0)
    o2_ref[...] = h[:, 0:8]
    o3_ref[...] = h[:, 8:16]
    o4_ref[...] = h[:, 16:24]


def _tc3_body(h2_ref, h3_ref, h4_ref, wo_ref, bo_ref, fcw_ref, fcb_ref, out_ref):
    o2 = jnp.maximum(jnp.dot(h2_ref[...], wo_ref[0], preferred_element_type=_f32) + bo_ref[0:1, :], 0.0)
    o3 = jnp.maximum(jnp.dot(h3_ref[...], wo_ref[1], preferred_element_type=_f32) + bo_ref[1:2, :], 0.0)
    o4 = jnp.maximum(jnp.dot(h4_ref[...], wo_ref[2], preferred_element_type=_f32) + bo_ref[2:3, :], 0.0)
    feat = jnp.concatenate([o2, o3, o4], axis=1)
    out_ref[...] = jnp.dot(feat, fcw_ref[...], preferred_element_type=_f32) + fcb_ref[...]


def _tc_call(body, n_out, *args):
    outs = [jax.ShapeDtypeStruct((N_PAD, W), _f32)] * n_out
    return pl.pallas_call(body, out_shape=outs)(*args)


_MESH = plsc.VectorSubcoreMesh(core_axis_name="c", subcore_axis_name="s")


@functools.partial(
    pl.kernel,
    out_type=(jax.ShapeDtypeStruct((N_PAD, W), _f32),) * 3,
    mesh=_MESH,
    scratch_types=[
        pltpu.VMEM_SHARED((N_PAD, W), _f32),
        pltpu.VMEM((NBLK, BLK * LANES), jnp.int32),
        pltpu.VMEM((NBLK, BLK * LANES), jnp.int32),
        pltpu.VMEM((BLK * LANES, W), _f32),
        pltpu.VMEM((BLK * LANES, W), _f32),
        pltpu.VMEM((BLK * LANES, W), _f32),
        pltpu.VMEM((BLK * LANES, W), _f32),
        pltpu.SemaphoreType.DMA,
        pltpu.SemaphoreType.DMA,
    ],
    compiler_params=pltpu.CompilerParams(use_tc_tiling_on_sc=False),
)
def _mp_kernel(h2_hbm, h3_hbm, h4_hbm,
               s2_hbm, d2_hbm, s3_hbm, d3_hbm, s4_hbm, d4_hbm,
               zeros_hbm,
               out2_hbm, out3_hbm, out4_hbm,
               acc, src_v, dst_v, buf0, buf1, buf2, buf3, sem_g, sem_s):
    cid = lax.axis_index("c")
    sid = lax.axis_index("s")
    blk_lo = sid * NBLK
    tab_lo = sid * SLICE
    bufs = [buf0, buf1, buf2, buf3]

    def my_tab(ref):
        return ref.at[pl.ds(tab_lo, SLICE)]

    pltpu.sync_copy(my_tab(zeros_hbm), my_tab(acc))
    plsc.subcore_barrier()

    def wait_one(sem):
        pltpu.make_async_copy(zeros_hbm.at[pl.ds(0, BLK * LANES)], buf0, sem).wait()

    def run_pattern(h0_hbm, src_hbm, dst_hbm, out_hbm, length):
        pltpu.sync_copy(src_hbm.at[pl.ds(blk_lo, NBLK)], src_v)
        pltpu.sync_copy(dst_hbm.at[pl.ds(blk_lo, NBLK)], dst_v)
        for r in range(length):
            cur = h0_hbm if r == 0 else out_hbm

            def gidx(i):
                return src_v.at[i]

            def sidx(i):
                return dst_v.at[i]

            pltpu.async_copy(cur.at[gidx(0)], bufs[0], sem_g)
            pltpu.async_copy(cur.at[gidx(1)], bufs[1], sem_g)

            def group_body(g, _):
                for di in range(4):
                    i = 4 * g + di
                    buf = bufs[di]
                    wait_one(sem_g)
                    pltpu.async_copy(buf, acc.at[sidx(i)], sem_s, add=True)
                    @pl.when(i >= 2)
                    def _():
                        wait_one(sem_s)

                    nxt = bufs[(di + 2) % 4]

                    @pl.when(i + 2 < NBLK)
                    def _():
                        pltpu.async_copy(cur.at[gidx(i + 2)], nxt, sem_g)
                return 0

            lax.fori_loop(0, NBLK // 4, group_body, 0)
            wait_one(sem_s)
            wait_one(sem_s)
            plsc.subcore_barrier()
            pltpu.sync_copy(my_tab(acc), my_tab(out_hbm))
            pltpu.sync_copy(my_tab(zeros_hbm), my_tab(acc))
            plsc.subcore_barrier()

    @pl.when(cid == 0)
    def _():
        run_pattern(h4_hbm, s4_hbm, d4_hbm, out4_hbm, 4)
        plsc.subcore_barrier()
        plsc.subcore_barrier()

    @pl.when(cid == 1)
    def _():
        run_pattern(h2_hbm, s2_hbm, d2_hbm, out2_hbm, 2)
        run_pattern(h3_hbm, s3_hbm, d3_hbm, out3_hbm, 3)


def _prep_edges(edge_index):
    src = jnp.full((E_PAD,), N, jnp.int32).at[:E].set(edge_index[0])
    dst = jnp.full((E_PAD,), N, jnp.int32).at[:E].set(edge_index[1])
    shp = (NSUB * NBLK, BLK * LANES)
    return src.reshape(shp), dst.reshape(shp)


def _pad_cols(w, total):
    return jnp.pad(w, ((0, 0), (0, total - w.shape[1])))


def _pad_rows(w, total):
    return jnp.pad(w, ((0, total - w.shape[0]), (0, 0)))


def kernel(batch, edge_index_c2, edge_index_c3, edge_index_c4, fc_b, fc_w,
           l0_c2_b_in, l0_c2_b_out, l0_c2_w_in, l0_c2_w_out,
           l0_c3_b_in, l0_c3_b_out, l0_c3_w_in, l0_c3_w_out,
           l0_c4_b_in, l0_c4_b_out, l0_c4_w_in, l0_c4_w_out,
           l1_c2_b_in, l1_c2_b_out, l1_c2_w_in, l1_c2_w_out,
           l1_c3_b_in, l1_c3_b_out, l1_c3_w_in, l1_c3_w_out,
           l1_c4_b_in, l1_c4_b_out, l1_c4_w_in, l1_c4_w_out,
           x):
    del batch

    x_pad = jnp.pad(x, ((0, N_PAD - N), (0, 0)))
    zeros = jnp.zeros((N_PAD, W), _f32)

    s2, d2 = _prep_edges(edge_index_c2)
    s3, d3 = _prep_edges(edge_index_c3)
    s4, d4 = _prep_edges(edge_index_c4)

    w0 = jnp.concatenate([_pad_cols(l0_c2_w_in, W),
                          _pad_cols(l0_c3_w_in, W),
                          _pad_cols(l0_c4_w_in, W)], axis=1)
    b0 = jnp.concatenate([jnp.pad(l0_c2_b_in, (0, W - 5)),
                          jnp.pad(l0_c3_b_in, (0, W - 5)),
                          jnp.pad(l0_c4_b_in, (0, W - 5))]).reshape(1, 3 * W)
    h2, h3, h4 = _tc_call(_tc1_body, 3, x_pad, w0, b0)

    m2, m3, m4 = _mp_kernel(h2, h3, h4, s2, d2, s3, d3, s4, d4, zeros)

    wo0 = jnp.stack([_pad_rows(l0_c2_w_out, W),
                     _pad_rows(l0_c3_w_out, W),
                     _pad_rows(l0_c4_w_out, W)])
    bo0 = jnp.stack([l0_c2_b_out, l0_c3_b_out, l0_c4_b_out])
    wi1 = jnp.concatenate([_pad_cols(l1_c2_w_in, W),
                           _pad_cols(l1_c3_w_in, W),
                           _pad_cols(l1_c4_w_in, W)], axis=1)
    bi1 = jnp.concatenate([jnp.pad(l1_c2_b_in, (0, W - 5)),
                           jnp.pad(l1_c3_b_in, (0, W - 5)),
                           jnp.pad(l1_c4_b_in, (0, W - 5))]).reshape(1, 3 * W)
    g2, g3, g4 = pl.pallas_call(
        _tc2_body, out_shape=[jax.ShapeDtypeStruct((N_PAD, W), _f32)] * 3,
    )(m2, m3, m4, wo0, bo0, wi1, bi1)

    n2, n3, n4 = _mp_kernel(g2, g3, g4, s2, d2, s3, d3, s4, d4, zeros)

    wo1 = jnp.stack([_pad_rows(l1_c2_w_out, W),
                     _pad_rows(l1_c3_w_out, W),
                     _pad_rows(l1_c4_w_out, W)])
    bo1 = jnp.stack([l1_c2_b_out, l1_c3_b_out, l1_c4_b_out])
    out = pl.pallas_call(
        _tc3_body, out_shape=jax.ShapeDtypeStruct((N_PAD, 10), _f32),
    )(n2, n3, n4, wo1, bo1, fc_w, fc_b.reshape(1, 10))
    return out[:N]


# speedup vs baseline: 28.3821x; 1.6095x over previous
import functools

import jax
import jax.numpy as jnp
from jax import lax
from jax.experimental import pallas as pl
from jax.experimental.pallas import tpu as pltpu
from jax.experimental.pallas import tpu_sc as plsc

N = 10000
E = 320000
NSUB = 16
EPS = E // NSUB
BLK = 1000
NBLK = EPS // BLK
N_PAD = 10112
SLICE = N_PAD // NSUB
W = 8

_f32 = jnp.float32


def _tc1_body(x_ref, w_ref, b_ref, o2_ref, o3_ref, o4_ref):
    h = jnp.dot(x_ref[...], w_ref[...], preferred_element_type=_f32)
    h = jnp.maximum(h + b_ref[...], 0.0)
    o2_ref[...] = h[:, 0:8]
    o3_ref[...] = h[:, 8:16]
    o4_ref[...] = h[:, 16:24]


def _tc2_body(h2_ref, h3_ref, h4_ref, wo_ref, bo_ref, wi_ref, bi_ref,
              o2_ref, o3_ref, o4_ref):
    o2 = jnp.maximum(jnp.dot(h2_ref[...], wo_ref[0], preferred_element_type=_f32) + bo_ref[0:1, :], 0.0)
    o3 = jnp.maximum(jnp.dot(h3_ref[...], wo_ref[1], preferred_element_type=_f32) + bo_ref[1:2, :], 0.0)
    o4 = jnp.maximum(jnp.dot(h4_ref[...], wo_ref[2], preferred_element_type=_f32) + bo_ref[2:3, :], 0.0)
    feat = jnp.concatenate([o2, o3, o4], axis=1)
    h = jnp.dot(feat, wi_ref[...], preferred_element_type=_f32)
    h = jnp.maximum(h + bi_ref[...], 0.0)
    o2_ref[...] = h[:, 0:8]
    o3_ref[...] = h[:, 8:16]
    o4_ref[...] = h[:, 16:24]


def _tc3_body(h2_ref, h3_ref, h4_ref, wo_ref, bo_ref, fcw_ref, fcb_ref, out_ref):
    o2 = jnp.maximum(jnp.dot(h2_ref[...], wo_ref[0], preferred_element_type=_f32) + bo_ref[0:1, :], 0.0)
    o3 = jnp.maximum(jnp.dot(h3_ref[...], wo_ref[1], preferred_element_type=_f32) + bo_ref[1:2, :], 0.0)
    o4 = jnp.maximum(jnp.dot(h4_ref[...], wo_ref[2], preferred_element_type=_f32) + bo_ref[2:3, :], 0.0)
    feat = jnp.concatenate([o2, o3, o4], axis=1)
    out_ref[...] = jnp.dot(feat, fcw_ref[...], preferred_element_type=_f32) + fcb_ref[...]


def _tc_call(body, n_out, *args):
    outs = [jax.ShapeDtypeStruct((N, W), _f32)] * n_out
    return pl.pallas_call(body, out_shape=outs)(*args)


_MESH = plsc.VectorSubcoreMesh(core_axis_name="c", subcore_axis_name="s")


@functools.partial(
    pl.kernel,
    out_type=(jax.ShapeDtypeStruct((N_PAD, W), _f32),) * 3,
    mesh=_MESH,
    scratch_types=[
        pltpu.VMEM_SHARED((N_PAD, W), _f32),
        pltpu.VMEM((NBLK, BLK), jnp.int32),
        pltpu.VMEM((NBLK, BLK), jnp.int32),
        pltpu.VMEM((BLK, W), _f32),
        pltpu.VMEM((BLK, W), _f32),
        pltpu.VMEM((BLK, W), _f32),
        pltpu.VMEM((BLK, W), _f32),
        pltpu.SemaphoreType.DMA,
        pltpu.SemaphoreType.DMA,
    ],
    compiler_params=pltpu.CompilerParams(use_tc_tiling_on_sc=False),
)
def _mp_kernel(h2_hbm, h3_hbm, h4_hbm,
               e2_hbm, e3_hbm, e4_hbm,
               zeros_hbm,
               out2_hbm, out3_hbm, out4_hbm,
               acc, src_v, dst_v, buf0, buf1, buf2, buf3, sem_g, sem_s):
    cid = lax.axis_index("c")
    sid = lax.axis_index("s")
    blk_lo = sid * NBLK
    tab_lo = sid * SLICE
    bufs = [buf0, buf1, buf2, buf3]

    def my_tab(ref):
        return ref.at[pl.ds(tab_lo, SLICE)]

    pltpu.sync_copy(my_tab(zeros_hbm), my_tab(acc))
    plsc.subcore_barrier()

    def wait_one(sem):
        pltpu.make_async_copy(zeros_hbm.at[pl.ds(0, BLK)], buf0, sem).wait()

    def run_pattern(h0_hbm, edges_hbm, out_hbm, length):
        pltpu.sync_copy(edges_hbm.at[0].at[pl.ds(blk_lo, NBLK)], src_v)
        pltpu.sync_copy(edges_hbm.at[1].at[pl.ds(blk_lo, NBLK)], dst_v)
        for r in range(length):
            cur = h0_hbm if r == 0 else out_hbm

            def gidx(i):
                return src_v.at[i]

            def sidx(i):
                return dst_v.at[i]

            pltpu.async_copy(cur.at[gidx(0)], bufs[0], sem_g)
            pltpu.async_copy(cur.at[gidx(1)], bufs[1], sem_g)

            def group_body(g, _):
                for di in range(4):
                    i = 4 * g + di
                    buf = bufs[di]
                    wait_one(sem_g)
                    pltpu.async_copy(buf, acc.at[sidx(i)], sem_s, add=True)
                    @pl.when(i >= 2)
                    def _():
                        wait_one(sem_s)

                    nxt = bufs[(di + 2) % 4]

                    @pl.when(i + 2 < NBLK)
                    def _():
                        pltpu.async_copy(cur.at[gidx(i + 2)], nxt, sem_g)
                return 0

            lax.fori_loop(0, NBLK // 4, group_body, 0)
            wait_one(sem_s)
            wait_one(sem_s)
            plsc.subcore_barrier()
            pltpu.sync_copy(my_tab(acc), my_tab(out_hbm))
            pltpu.sync_copy(my_tab(zeros_hbm), my_tab(acc))
            plsc.subcore_barrier()

    @pl.when(cid == 0)
    def _():
        run_pattern(h4_hbm, e4_hbm, out4_hbm, 4)
        plsc.subcore_barrier()
        plsc.subcore_barrier()

    @pl.when(cid == 1)
    def _():
        run_pattern(h2_hbm, e2_hbm, out2_hbm, 2)
        run_pattern(h3_hbm, e3_hbm, out3_hbm, 3)


def _prep_edges(edge_index):
    return edge_index.reshape(2, NSUB * NBLK, BLK)


def _pad_cols(w, total):
    return jnp.pad(w, ((0, 0), (0, total - w.shape[1])))


def _pad_rows(w, total):
    return jnp.pad(w, ((0, total - w.shape[0]), (0, 0)))


def kernel(batch, edge_index_c2, edge_index_c3, edge_index_c4, fc_b, fc_w,
           l0_c2_b_in, l0_c2_b_out, l0_c2_w_in, l0_c2_w_out,
           l0_c3_b_in, l0_c3_b_out, l0_c3_w_in, l0_c3_w_out,
           l0_c4_b_in, l0_c4_b_out, l0_c4_w_in, l0_c4_w_out,
           l1_c2_b_in, l1_c2_b_out, l1_c2_w_in, l1_c2_w_out,
           l1_c3_b_in, l1_c3_b_out, l1_c3_w_in, l1_c3_w_out,
           l1_c4_b_in, l1_c4_b_out, l1_c4_w_in, l1_c4_w_out,
           x):
    del batch

    zeros = jnp.zeros((N_PAD, W), _f32)

    e2 = _prep_edges(edge_index_c2)
    e3 = _prep_edges(edge_index_c3)
    e4 = _prep_edges(edge_index_c4)

    w0 = jnp.concatenate([_pad_cols(l0_c2_w_in, W),
                          _pad_cols(l0_c3_w_in, W),
                          _pad_cols(l0_c4_w_in, W)], axis=1)
    b0 = jnp.concatenate([jnp.pad(l0_c2_b_in, (0, W - 5)),
                          jnp.pad(l0_c3_b_in, (0, W - 5)),
                          jnp.pad(l0_c4_b_in, (0, W - 5))]).reshape(1, 3 * W)
    h2, h3, h4 = _tc_call(_tc1_body, 3, x, w0, b0)

    m2, m3, m4 = _mp_kernel(h2, h3, h4, e2, e3, e4, zeros)

    wo0 = jnp.stack([_pad_rows(l0_c2_w_out, W),
                     _pad_rows(l0_c3_w_out, W),
                     _pad_rows(l0_c4_w_out, W)])
    bo0 = jnp.stack([l0_c2_b_out, l0_c3_b_out, l0_c4_b_out])
    wi1 = jnp.concatenate([_pad_cols(l1_c2_w_in, W),
                           _pad_cols(l1_c3_w_in, W),
                           _pad_cols(l1_c4_w_in, W)], axis=1)
    bi1 = jnp.concatenate([jnp.pad(l1_c2_b_in, (0, W - 5)),
                           jnp.pad(l1_c3_b_in, (0, W - 5)),
                           jnp.pad(l1_c4_b_in, (0, W - 5))]).reshape(1, 3 * W)
    g2, g3, g4 = pl.pallas_call(
        _tc2_body, out_shape=[jax.ShapeDtypeStruct((N_PAD, W), _f32)] * 3,
    )(m2, m3, m4, wo0, bo0, wi1, bi1)

    n2, n3, n4 = _mp_kernel(g2, g3, g4, e2, e3, e4, zeros)

    wo1 = jnp.stack([_pad_rows(l1_c2_w_out, W),
                     _pad_rows(l1_c3_w_out, W),
                     _pad_rows(l1_c4_w_out, W)])
    bo1 = jnp.stack([l1_c2_b_out, l1_c3_b_out, l1_c4_b_out])
    out = pl.pallas_call(
        _tc3_body, out_shape=jax.ShapeDtypeStruct((N_PAD, 10), _f32),
    )(n2, n3, n4, wo1, bo1, fc_w, fc_b.reshape(1, 10))
    return out[:N]
